# baseline (speedup 1.0000x reference)
import functools

import jax
import jax.numpy as jnp
from jax import lax
from jax.experimental import pallas as pl
from jax.experimental.pallas import tpu as pltpu
from jax.experimental.pallas import tpu_sc as plsc

_mesh = plsc.VectorSubcoreMesh(core_axis_name="c", subcore_axis_name="s")


@functools.partial(
    pl.kernel,
    out_type=jax.ShapeDtypeStruct((16,), jnp.float32),
    mesh=_mesh,
    scratch_types=[
        pltpu.VMEM((16,), jnp.float32),
        pltpu.SemaphoreType.DMA,
    ],
)
def _sc_tiny(out_hbm, zbuf, sem):
    wid = lax.axis_index("s") * 2 + lax.axis_index("c")

    @pl.when(wid == 0)
    def _():
        zbuf[...] = jnp.zeros((16,), jnp.float32)
        pltpu.sync_copy(zbuf, out_hbm)


def kernel(x):
    del x
    return _sc_tiny().reshape(1, 16)


# speedup vs baseline: 1.6026x; 1.6026x over previous
import jax
import jax.numpy as jnp
from jax.experimental import pallas as pl

_CHANNELS = 32768
_ROWS = 128
_BLOCK_ROWS = 64
_CH = 8192
_NC = _CHANNELS // _CH


def _onehot_argmax_kernel(x_ref, o_ref):
    ms = []
    for c in range(_NC):
        ms.append(jnp.max(x_ref[:, c * _CH:(c + 1) * _CH], axis=1, keepdims=True))
    m = jnp.maximum(jnp.maximum(ms[0], ms[1]), jnp.maximum(ms[2], ms[3]))
    idxs = []
    for c in range(_NC):
        xc = x_ref[:, c * _CH:(c + 1) * _CH]
        col = jax.lax.broadcasted_iota(jnp.int32, xc.shape, 1) + c * _CH
        idxs.append(jnp.min(jnp.where(xc == m, col, _CHANNELS), axis=1, keepdims=True))
    idx = jnp.minimum(jnp.minimum(idxs[0], idxs[1]), jnp.minimum(idxs[2], idxs[3]))
    for c in range(_NC):
        col = jax.lax.broadcasted_iota(jnp.int32, (_BLOCK_ROWS, _CH), 1) + c * _CH
        o_ref[:, c * _CH:(c + 1) * _CH] = (col == idx).astype(jnp.float32)


def kernel(x):
    return pl.pallas_call(
        _onehot_argmax_kernel,
        grid=(_ROWS // _BLOCK_ROWS,),
        in_specs=[pl.BlockSpec((_BLOCK_ROWS, _CHANNELS), lambda i: (i, 0))],
        out_specs=pl.BlockSpec((_BLOCK_ROWS, _CHANNELS), lambda i: (i, 0)),
        out_shape=jax.ShapeDtypeStruct((_ROWS, _CHANNELS), jnp.float32),
    )(x)


# speedup vs baseline: 1.6077x; 1.0031x over previous
import functools as _ft

import jax
import jax.numpy as jnp
from jax.experimental import pallas as pl

_CHANNELS = 32768
_ROWS = 128
_BLOCK_ROWS = 64
_CH = 8192
_NC = _CHANNELS // _CH


def _onehot_argmax_kernel(x_ref, o_ref):
    ms = []
    for c in range(_NC):
        ms.append(jnp.max(x_ref[:, c * _CH:(c + 1) * _CH], axis=1, keepdims=True))
    m = _ft.reduce(jnp.maximum, ms)
    idxs = []
    for c in range(_NC):
        xc = x_ref[:, c * _CH:(c + 1) * _CH]
        col = jax.lax.broadcasted_iota(jnp.int32, xc.shape, 1) + c * _CH
        idxs.append(jnp.min(jnp.where(xc == m, col, _CHANNELS), axis=1, keepdims=True))
    idx = _ft.reduce(jnp.minimum, idxs)
    for c in range(_NC):
        col = jax.lax.broadcasted_iota(jnp.int32, (_BLOCK_ROWS, _CH), 1) + c * _CH
        o_ref[:, c * _CH:(c + 1) * _CH] = (col == idx).astype(jnp.float32)


def kernel(x):
    return pl.pallas_call(
        _onehot_argmax_kernel,
        grid=(_ROWS // _BLOCK_ROWS,),
        in_specs=[pl.BlockSpec((_BLOCK_ROWS, _CHANNELS), lambda i: (i, 0))],
        out_specs=pl.BlockSpec((_BLOCK_ROWS, _CHANNELS), lambda i: (i, 0)),
        out_shape=jax.ShapeDtypeStruct((_ROWS, _CHANNELS), jnp.float32),
    )(x)
